# baseline (speedup 1.0000x reference)
import jax
import jax.numpy as jnp
from jax.experimental import pallas as pl
from jax.experimental.pallas import tpu as pltpu

N = 4096
F = 256
L = 5532
Q = 5000
SCALAR = 30.0
BC = 512
NLB = (L + BC - 1) // BC
NQB = (Q + BC - 1) // BC
NB = NLB + NQB
NEG = -1e30


def _oim_body(x_ref, lut_ref, cq_ref, lbl_ref, out_ref, xb_s, m_s, s_s, g_s):
    j = pl.program_id(0)

    @pl.when(j == 0)
    def _init():
        xb_s[...] = x_ref[...].astype(jnp.bfloat16)
        m_s[...] = jnp.full((N, 1), NEG, dtype=jnp.float32)
        s_s[...] = jnp.zeros((N, 1), dtype=jnp.float32)
        g_s[...] = jnp.zeros((N, 1), dtype=jnp.float32)

    def update(t_ref, base, limit, do_mask, do_hit):
        t = t_ref[...].astype(jnp.bfloat16)
        logits = SCALAR * jax.lax.dot_general(
            xb_s[...], t, (((1,), (1,)), ((), ())),
            preferred_element_type=jnp.float32)
        if do_mask or do_hit:
            col = base + jax.lax.broadcasted_iota(jnp.int32, (1, BC), 1)
        if do_mask:
            logits = jnp.where(col < limit, logits, NEG)
        if do_hit:
            hit = col == lbl_ref[...].astype(jnp.int32)
            g_s[...] += jnp.sum(jnp.where(hit, logits, 0.0),
                                axis=1, keepdims=True)
        m_old = m_s[...]
        m_new = jnp.maximum(m_old, jnp.max(logits, axis=1, keepdims=True))
        s_s[...] = (s_s[...] * jnp.exp(m_old - m_new)
                    + jnp.sum(jnp.exp(logits - m_new), axis=1, keepdims=True))
        m_s[...] = m_new

    @pl.when(j < NLB - 1)
    def _lut_body():
        update(lut_ref, j * BC, L, do_mask=False, do_hit=True)

    @pl.when(j == NLB - 1)
    def _lut_tail():
        update(lut_ref, j * BC, L, do_mask=True, do_hit=True)

    @pl.when((j >= NLB) & (j < NB - 1))
    def _cq_body():
        update(cq_ref, L + (j - NLB) * BC, L + Q, do_mask=False, do_hit=False)

    @pl.when(j == NB - 1)
    def _cq_tail():
        update(cq_ref, L + (j - NLB) * BC, L + Q, do_mask=True, do_hit=False)
        valid = lbl_ref[...] >= 0.0
        nll = m_s[...] + jnp.log(s_s[...]) - g_s[...]
        loss_sum = jnp.sum(jnp.where(valid, nll, 0.0), keepdims=True)
        cnt = jnp.sum(valid.astype(jnp.float32), keepdims=True)
        out_ref[...] = loss_sum / jnp.maximum(cnt, 1.0)


@jax.jit
def _oim_loss(inputs, label_f, lut, cq):
    out = pl.pallas_call(
        _oim_body,
        grid=(NB,),
        in_specs=[
            pl.BlockSpec((N, F), lambda j: (0, 0)),
            pl.BlockSpec((BC, F), lambda j: (jnp.minimum(j, NLB - 1), 0)),
            pl.BlockSpec((BC, F), lambda j: (jnp.maximum(j - NLB, 0), 0)),
            pl.BlockSpec((N, 1), lambda j: (0, 0)),
        ],
        out_specs=pl.BlockSpec((1, 1), lambda j: (0, 0)),
        out_shape=jax.ShapeDtypeStruct((1, 1), jnp.float32),
        scratch_shapes=[
            pltpu.VMEM((N, F), jnp.bfloat16),
            pltpu.VMEM((N, 1), jnp.float32),
            pltpu.VMEM((N, 1), jnp.float32),
            pltpu.VMEM((N, 1), jnp.float32),
        ],
        compiler_params=pltpu.CompilerParams(
            dimension_semantics=("arbitrary",)),
    )(inputs, lut, cq, label_f)
    return out[0, 0]


def kernel(inputs, roi_label, detectionscore, lut, cq):
    label_f = (roi_label.reshape(-1, 1) - 1).astype(jnp.float32)
    loss = _oim_loss(inputs, label_f, lut, cq)
    return (loss, lut)


# speedup vs baseline: 1.0054x; 1.0054x over previous
import jax
import jax.numpy as jnp
from jax.experimental import pallas as pl
from jax.experimental.pallas import tpu as pltpu

N = 4096
F = 256
L = 5532
Q = 5000
SCALAR = 30.0
BC = 512
NLB = (L + BC - 1) // BC
NQB = (Q + BC - 1) // BC
NB = NLB + NQB
NEG = -1e30


def _oim_body(x_ref, lut_ref, cq_ref, lbl_ref, out_ref, m_s, s_s, g_s):
    j = pl.program_id(0)

    @pl.when(j == 0)
    def _init():
        m_s[...] = jnp.full((N, 1), NEG, dtype=jnp.float32)
        s_s[...] = jnp.zeros((N, 1), dtype=jnp.float32)
        g_s[...] = jnp.zeros((N, 1), dtype=jnp.float32)

    def update(t_ref, base, limit, do_mask, do_hit):
        logits = SCALAR * jax.lax.dot_general(
            x_ref[...], t_ref[...], (((1,), (1,)), ((), ())),
            preferred_element_type=jnp.float32)
        if do_mask or do_hit:
            col = base + jax.lax.broadcasted_iota(jnp.int32, (1, BC), 1)
        if do_mask:
            logits = jnp.where(col < limit, logits, NEG)
        if do_hit:
            hit = col == lbl_ref[...].astype(jnp.int32)
            g_s[...] += jnp.sum(jnp.where(hit, logits, 0.0),
                                axis=1, keepdims=True)
        m_old = m_s[...]
        m_new = jnp.maximum(m_old, jnp.max(logits, axis=1, keepdims=True))
        s_s[...] = (s_s[...] * jnp.exp(m_old - m_new)
                    + jnp.sum(jnp.exp(logits - m_new), axis=1, keepdims=True))
        m_s[...] = m_new

    @pl.when(j < NLB - 1)
    def _lut_body():
        update(lut_ref, j * BC, L, do_mask=False, do_hit=True)

    @pl.when(j == NLB - 1)
    def _lut_tail():
        update(lut_ref, j * BC, L, do_mask=True, do_hit=True)

    @pl.when((j >= NLB) & (j < NB - 1))
    def _cq_body():
        update(cq_ref, L + (j - NLB) * BC, L + Q, do_mask=False, do_hit=False)

    @pl.when(j == NB - 1)
    def _cq_tail():
        update(cq_ref, L + (j - NLB) * BC, L + Q, do_mask=True, do_hit=False)
        valid = lbl_ref[...] >= 0.0
        nll = m_s[...] + jnp.log(s_s[...]) - g_s[...]
        loss_sum = jnp.sum(jnp.where(valid, nll, 0.0), keepdims=True)
        cnt = jnp.sum(valid.astype(jnp.float32), keepdims=True)
        out_ref[...] = loss_sum / jnp.maximum(cnt, 1.0)


@jax.jit
def _oim_loss(inputs, label_f, lut, cq):
    out = pl.pallas_call(
        _oim_body,
        grid=(NB,),
        in_specs=[
            pl.BlockSpec((N, F), lambda j: (0, 0)),
            pl.BlockSpec((BC, F), lambda j: (jnp.minimum(j, NLB - 1), 0)),
            pl.BlockSpec((BC, F), lambda j: (jnp.maximum(j - NLB, 0), 0)),
            pl.BlockSpec((N, 1), lambda j: (0, 0)),
        ],
        out_specs=pl.BlockSpec((1, 1), lambda j: (0, 0)),
        out_shape=jax.ShapeDtypeStruct((1, 1), jnp.float32),
        scratch_shapes=[
            pltpu.VMEM((N, 1), jnp.float32),
            pltpu.VMEM((N, 1), jnp.float32),
            pltpu.VMEM((N, 1), jnp.float32),
        ],
        compiler_params=pltpu.CompilerParams(
            dimension_semantics=("arbitrary",)),
    )(inputs, lut, cq, label_f)
    return out[0, 0]


def kernel(inputs, roi_label, detectionscore, lut, cq):
    label_f = (roi_label.reshape(-1, 1) - 1).astype(jnp.float32)
    loss = _oim_loss(inputs, label_f, lut, cq)
    return (loss, lut)


# speedup vs baseline: 1.2423x; 1.2356x over previous
import jax
import jax.numpy as jnp
from jax.experimental import pallas as pl
from jax.experimental.pallas import tpu as pltpu

N = 4096
F = 256
L = 5532
Q = 5000
SCALAR = 30.0
BC = 512
NLB = (L + BC - 1) // BC
NQB = (Q + BC - 1) // BC
NB = NLB + NQB
NEG = -1e30


def _oim_body(x_ref, lut_ref, cq_ref, lbl_ref, out_ref, t_s, m_s, s_s, g_s):
    j = pl.program_id(0)

    @pl.when(j == 0)
    def _init():
        m_s[...] = jnp.full((N, 1), NEG, dtype=jnp.float32)
        s_s[...] = jnp.zeros((N, 1), dtype=jnp.float32)
        g_s[...] = jnp.zeros((N, 1), dtype=jnp.float32)

    is_lut = j < NLB

    @pl.when(is_lut)
    def _pick_lut():
        t_s[...] = lut_ref[...]

    @pl.when(jnp.logical_not(is_lut))
    def _pick_cq():
        t_s[...] = cq_ref[...]

    logits = SCALAR * jax.lax.dot_general(
        x_ref[...], t_s[...], (((1,), (1,)), ((), ())),
        preferred_element_type=jnp.float32)

    base = jnp.where(is_lut, j * BC, L + (j - NLB) * BC)
    limit = jnp.where(is_lut, L, L + Q)
    col = base + jax.lax.broadcasted_iota(jnp.int32, (1, BC), 1)
    masked = jnp.where(col < limit, logits, NEG)

    lbl = lbl_ref[...].astype(jnp.int32)
    hit = col == lbl
    g_s[...] += jnp.sum(jnp.where(hit, masked, 0.0), axis=1, keepdims=True)

    m_old = m_s[...]
    m_new = jnp.maximum(m_old, jnp.max(masked, axis=1, keepdims=True))
    s_s[...] = (s_s[...] * jnp.exp(m_old - m_new)
                + jnp.sum(jnp.exp(masked - m_new), axis=1, keepdims=True))
    m_s[...] = m_new

    @pl.when(j == NB - 1)
    def _finish():
        valid = lbl_ref[...] >= 0.0
        nll = m_s[...] + jnp.log(s_s[...]) - g_s[...]
        loss_sum = jnp.sum(jnp.where(valid, nll, 0.0), keepdims=True)
        cnt = jnp.sum(valid.astype(jnp.float32), keepdims=True)
        out_ref[...] = loss_sum / jnp.maximum(cnt, 1.0)


@jax.jit
def _oim_loss(inputs, label_f, lut, cq):
    out = pl.pallas_call(
        _oim_body,
        grid=(NB,),
        in_specs=[
            pl.BlockSpec((N, F), lambda j: (0, 0)),
            pl.BlockSpec((BC, F), lambda j: (jnp.minimum(j, NLB - 1), 0)),
            pl.BlockSpec((BC, F), lambda j: (jnp.maximum(j - NLB, 0), 0)),
            pl.BlockSpec((N, 1), lambda j: (0, 0)),
        ],
        out_specs=pl.BlockSpec((1, 1), lambda j: (0, 0)),
        out_shape=jax.ShapeDtypeStruct((1, 1), jnp.float32),
        scratch_shapes=[
            pltpu.VMEM((BC, F), jnp.float32),
            pltpu.VMEM((N, 1), jnp.float32),
            pltpu.VMEM((N, 1), jnp.float32),
            pltpu.VMEM((N, 1), jnp.float32),
        ],
        compiler_params=pltpu.CompilerParams(
            dimension_semantics=("arbitrary",)),
    )(inputs, lut, cq, label_f)
    return out[0, 0]


def kernel(inputs, roi_label, detectionscore, lut, cq):
    label_f = (roi_label.reshape(-1, 1) - 1).astype(jnp.float32)
    loss = _oim_loss(inputs, label_f, lut, cq)
    return (loss, lut)
